# baseline (speedup 1.0000x reference)
import jax
import jax.numpy as jnp
from jax.experimental import pallas as pl
from jax.experimental.pallas import tpu as pltpu
from jax.experimental.pallas import tpu_sc as plsc


BLK_S = 512
SC_BLK_S = 16
SC_LANES = 16


def _tc_add_kernel(x_ref, pos_ref, o_ref):
    o_ref[...] = x_ref[...] + pos_ref[...][None, :, :]


def _tc_part(x, pos_table, nb):
    B, S, D = x.shape
    return pl.pallas_call(
        _tc_add_kernel,
        grid=(S // BLK_S,),
        in_specs=[
            pl.BlockSpec((nb, BLK_S, D), lambda i: (0, i, 0)),
            pl.BlockSpec((BLK_S, D), lambda i: (i, 0)),
        ],
        out_specs=pl.BlockSpec((nb, BLK_S, D), lambda i: (0, i, 0)),
        out_shape=jax.ShapeDtypeStruct((nb, S, D), x.dtype),
    )(x, pos_table)


def _sc_part(x, pos_table, b):
    B, S, D = x.shape
    mesh = plsc.VectorSubcoreMesh(core_axis_name="core",
                                  subcore_axis_name="subcore")

    @pl.kernel(out_type=jax.ShapeDtypeStruct((1, S, D), x.dtype), mesh=mesh)
    def sc_kernel(x_hbm, pos_hbm, o_hbm):
        def body(x_vmem, pos_vmem, o_vmem):
            @pl.loop(0, SC_BLK_S)
            def _(r):
                @pl.loop(0, D, step=4 * SC_LANES)
                def _(c):
                    for u in range(4):
                        slc = (pl.ds(r, 1), pl.ds(c + u * SC_LANES, SC_LANES))
                        o_vmem.at[*slc][...] = (
                            x_vmem.at[*slc][...] + pos_vmem.at[*slc][...]
                        )

        pltpu.emit_pipeline(
            body,
            grid=(S // SC_BLK_S,),
            in_specs=[
                pl.BlockSpec((SC_BLK_S, D), index_map=lambda i: (i, 0)),
                pl.BlockSpec((SC_BLK_S, D), index_map=lambda i: (i, 0)),
            ],
            out_specs=[pl.BlockSpec((SC_BLK_S, D), index_map=lambda i: (i, 0))],
            core_axis_name=("core", "subcore"),
            dimension_semantics=(pltpu.PARALLEL,),
        )(x_hbm.at[b], pos_hbm, o_hbm.at[0])

    return sc_kernel(x, pos_table)


def kernel(x, pos_table):
    B, S, D = x.shape
    tc_out = _tc_part(x, pos_table, B - 1)
    sc_out = _sc_part(x, pos_table, B - 1)
    return jnp.concatenate([tc_out, sc_out], axis=0)


# speedup vs baseline: 2.2418x; 2.2418x over previous
import jax
import jax.numpy as jnp
from jax.experimental import pallas as pl


BLK_S = 2048


def _add_kernel(x_ref, pos_ref, o_ref):
    o_ref[...] = x_ref[...] + pos_ref[...][None, :, :]


def kernel(x, pos_table):
    B, S, D = x.shape
    grid = (S // BLK_S, B)
    return pl.pallas_call(
        _add_kernel,
        grid=grid,
        in_specs=[
            pl.BlockSpec((1, BLK_S, D), lambda i, b: (b, i, 0)),
            pl.BlockSpec((BLK_S, D), lambda i, b: (i, 0)),
        ],
        out_specs=pl.BlockSpec((1, BLK_S, D), lambda i, b: (b, i, 0)),
        out_shape=jax.ShapeDtypeStruct((B, S, D), x.dtype),
    )(x, pos_table)
